# baseline (speedup 1.0000x reference)
import functools
import math

import jax
import jax.numpy as jnp
from jax import lax
from jax.experimental import pallas as pl
from jax.experimental.pallas import tpu as pltpu
from jax.experimental.pallas import tpu_sc as plsc

N = 10000
E = 320000
D = 128
H = 128
L = 128
B = 512

CORES = 2
SUB = 16
NW = CORES * SUB

N_PAD = 10240
SLAB = N_PAD // SUB
N_TAB = 10112
TSLAB = N_TAB // SUB
CHUNK = 128
E_PAD = 331776
EC = E_PAD // SUB // CHUNK
DC = E_PAD // NW // CHUNK
B_PAD = 520
PW = 80
PC = 64
PPW = N_PAD // NW
PCH = PPW // PC

_MESH = dict(mesh=plsc.VectorSubcoreMesh(core_axis_name="c", subcore_axis_name="s"))
_F32 = jnp.float32
_INV_BN = 1.0 / math.sqrt(1.0 + 1e-5)
_PREC = jax.lax.Precision.HIGHEST


@functools.partial(
    pl.kernel,
    out_type=jax.ShapeDtypeStruct((2, CORES, N_PAD, 16), _F32),
    scratch_types=[
        pltpu.VMEM_SHARED((N_PAD, 16), _F32),
        pltpu.VMEM_SHARED((N_PAD, 16), _F32),
        pltpu.VMEM((CHUNK,), jnp.int32),
        pltpu.VMEM((CHUNK,), jnp.int32),
        pltpu.VMEM((CHUNK,), jnp.int32),
        pltpu.VMEM((CHUNK,), jnp.int32),
        pltpu.VMEM((CHUNK,), jnp.int32),
        pltpu.VMEM((CHUNK,), jnp.int32),
        pltpu.VMEM((CHUNK, 16), _F32),
        pltpu.SemaphoreType.DMA,
        pltpu.SemaphoreType.DMA,
        pltpu.SemaphoreType.DMA,
        pltpu.SemaphoreType.DMA,
        pltpu.SemaphoreType.DMA,
        pltpu.SemaphoreType.DMA,
    ],
    **_MESH,
)
def _deg_kernel(dst_hbm, ones_hbm, deg_out, degA, degB,
                a0, a1, a2, b0, b1, b2, onesbuf, i0, i1, i2, t0, t1, t2):
    c = lax.axis_index("c")
    s = lax.axis_index("s")
    wid = c * SUB + s
    Ea = (a0, a1, a2)
    Eb = (b0, b1, b2)
    semI = (i0, i1, i2)
    semS = (t0, t1, t2)
    pltpu.sync_copy(ones_hbm.at[pl.ds(0, CHUNK), :], onesbuf)
    pltpu.sync_copy(ones_hbm.at[pl.ds(s * SLAB, SLAB), :], degA.at[pl.ds(s * SLAB, SLAB)])
    pltpu.sync_copy(ones_hbm.at[pl.ds(s * SLAB, SLAB), :], degB.at[pl.ds(s * SLAB, SLAB)])
    plsc.subcore_barrier()

    def idx_load(j, k):
        jw = jnp.where(j < DC, j, 0)
        pltpu.async_copy(dst_hbm.at[0, wid, jw], Ea[k], semI[k])
        pltpu.async_copy(dst_hbm.at[1, wid, jw], Eb[k], semI[k])

    def wait_idx(k):
        pltpu.make_async_copy(dst_hbm.at[0, wid, 0], Ea[k], semI[k]).wait()
        pltpu.make_async_copy(dst_hbm.at[1, wid, 0], Eb[k], semI[k]).wait()

    def wait_scat(k):
        pltpu.make_async_copy(onesbuf, degA.at[Ea[k]], semS[k]).wait()
        pltpu.make_async_copy(onesbuf, degB.at[Eb[k]], semS[k]).wait()

    def step(j, jj, first):
        X, Z = jj % 3, (jj + 2) % 3
        if not first:
            wait_idx(X)
        pltpu.async_copy(onesbuf, degA.at[Ea[X]], semS[X], add=True)
        pltpu.async_copy(onesbuf, degB.at[Eb[X]], semS[X], add=True)
        if not first:
            wait_scat(Z)
        idx_load(j + 2, Z)

    pltpu.sync_copy(dst_hbm.at[0, wid, 0], a0)
    pltpu.sync_copy(dst_hbm.at[1, wid, 0], b0)
    idx_load(jnp.int32(1), 1)
    step(jnp.int32(0), 0, True)
    step(jnp.int32(1), 1, False)
    step(jnp.int32(2), 2, False)

    def body(t, carry):
        j = t * 3
        step(j, 0, False)
        step(j + 1, 1, False)
        step(j + 2, 2, False)
        return carry

    lax.fori_loop(1, DC // 3, body, 0)
    wait_scat((DC - 1) % 3)
    wait_idx(DC % 3)
    wait_idx((DC + 1) % 3)
    plsc.subcore_barrier()
    pltpu.sync_copy(degA.at[pl.ds(s * SLAB, SLAB)], deg_out.at[0, c, pl.ds(s * SLAB, SLAB), :])
    pltpu.sync_copy(degB.at[pl.ds(s * SLAB, SLAB)], deg_out.at[1, c, pl.ds(s * SLAB, SLAB), :])


def _make_edge_kernel(P, m):
    @functools.partial(
        pl.kernel,
        out_type=jax.ShapeDtypeStruct((CORES, N_PAD, P), _F32),
        scratch_types=[
            pltpu.VMEM_SHARED((N_TAB, P), _F32),
            pltpu.VMEM_SHARED((N_TAB, P), _F32),
            pltpu.VMEM((CHUNK,), jnp.int32),
            pltpu.VMEM((CHUNK,), jnp.int32),
            pltpu.VMEM((CHUNK,), jnp.int32),
            pltpu.VMEM((CHUNK,), jnp.int32),
            pltpu.VMEM((CHUNK,), jnp.int32),
            pltpu.VMEM((CHUNK,), jnp.int32),
            pltpu.VMEM((CHUNK, P), _F32),
            pltpu.VMEM((CHUNK, P), _F32),
            pltpu.VMEM((CHUNK, P), _F32),
            pltpu.SemaphoreType.DMA,
            pltpu.SemaphoreType.DMA,
            pltpu.SemaphoreType.DMA,
            pltpu.SemaphoreType.DMA,
            pltpu.SemaphoreType.DMA,
            pltpu.SemaphoreType.DMA,
            pltpu.SemaphoreType.DMA,
            pltpu.SemaphoreType.DMA,
            pltpu.SemaphoreType.DMA,
        ],
        **_MESH,
    )
    def _edge_kernel(y_hbm, src_hbm, dst_hbm, z_out, ytab, ztab,
                     s0, s1, s2, d0, d1, d2, g0, g1, g2,
                     i0, i1, i2, q0, q1, q2, t0, t1, t2):
        c = lax.axis_index("c")
        s = lax.axis_index("s")
        S = (s0, s1, s2)
        Dd = (d0, d1, d2)
        Gb = (g0, g1, g2)
        semI = (i0, i1, i2)
        semG = (q0, q1, q2)
        semS = (t0, t1, t2)

        if True:
            pltpu.sync_copy(y_hbm.at[c, pl.ds(s * TSLAB, TSLAB), :], ytab.at[pl.ds(s * TSLAB, TSLAB)])
            pltpu.sync_copy(y_hbm.at[c, pl.ds(s * TSLAB, TSLAB), :], ztab.at[pl.ds(s * TSLAB, TSLAB)])
            plsc.subcore_barrier()

            def idx_load(j, k):
                jw = jnp.where(j < EC, j, 0)
                pltpu.async_copy(src_hbm.at[m, s, jw], S[k], semI[k])
                pltpu.async_copy(dst_hbm.at[m, s, jw], Dd[k], semI[k])

            def wait_idx(k):
                pltpu.make_async_copy(src_hbm.at[m, s, 0], S[k], semI[k]).wait()
                pltpu.make_async_copy(dst_hbm.at[m, s, 0], Dd[k], semI[k]).wait()

            def step(j, jj, first):
                X, Y, Z = jj % 3, (jj + 1) % 3, (jj + 2) % 3
                pltpu.make_async_copy(ytab.at[S[X]], Gb[X], semG[X]).wait()
                pltpu.async_copy(Gb[X], ztab.at[Dd[X]], semS[X], add=True)
                if first:
                    pass
                else:
                    pltpu.make_async_copy(Gb[Z], ztab.at[Dd[Z]], semS[Z]).wait()
                idx_load(j + 2, Z)
                wait_idx(Y)
                pltpu.async_copy(ytab.at[S[Y]], Gb[Y], semG[Y])

            pltpu.sync_copy(src_hbm.at[m, s, 0], s0)
            pltpu.sync_copy(dst_hbm.at[m, s, 0], d0)
            pltpu.async_copy(ytab.at[s0], g0, semG[0])
            idx_load(jnp.int32(1), 1)

            step(jnp.int32(0), 0, True)
            step(jnp.int32(1), 1, False)
            step(jnp.int32(2), 2, False)

            def body(t, carry):
                j = t * 3
                step(j, 0, False)
                step(j + 1, 1, False)
                step(j + 2, 2, False)
                return carry

            lax.fori_loop(1, EC // 3, body, 0)

            pltpu.make_async_copy(Gb[2], ztab.at[Dd[2]], semS[2]).wait()
            pltpu.make_async_copy(ytab.at[S[0]], Gb[0], semG[0]).wait()
            wait_idx(1)
            plsc.subcore_barrier()
            pltpu.sync_copy(ztab.at[pl.ds(s * TSLAB, TSLAB)], z_out.at[c, pl.ds(s * TSLAB, TSLAB), :])
            plsc.subcore_barrier()

    return _edge_kernel


_edge64_m = tuple(_make_edge_kernel(64, m) for m in range(2))

def _make_edge4_kernel(m):
    ECH = EC // 2

    @functools.partial(
        pl.kernel,
        out_type=jax.ShapeDtypeStruct((CORES, N_PAD, 64), _F32),
        scratch_types=[
            pltpu.VMEM_SHARED((N_TAB, 64), _F32),
            pltpu.VMEM_SHARED((N_TAB, 64), _F32),
            pltpu.VMEM((CHUNK,), jnp.int32),
            pltpu.VMEM((CHUNK,), jnp.int32),
            pltpu.VMEM((CHUNK,), jnp.int32),
            pltpu.VMEM((CHUNK,), jnp.int32),
            pltpu.VMEM((CHUNK,), jnp.int32),
            pltpu.VMEM((CHUNK,), jnp.int32),
            pltpu.VMEM((CHUNK, 64), _F32),
            pltpu.VMEM((CHUNK, 64), _F32),
            pltpu.VMEM((CHUNK, 64), _F32),
            pltpu.SemaphoreType.DMA,
            pltpu.SemaphoreType.DMA,
            pltpu.SemaphoreType.DMA,
            pltpu.SemaphoreType.DMA,
            pltpu.SemaphoreType.DMA,
            pltpu.SemaphoreType.DMA,
            pltpu.SemaphoreType.DMA,
            pltpu.SemaphoreType.DMA,
            pltpu.SemaphoreType.DMA,
        ],
        **_MESH,
    )
    def _edge4_kernel(y_hbm, src_hbm, dst_hbm, z_out, ytab, ztab,
                      s0, s1, s2, d0, d1, d2, g0, g1, g2,
                      i0, i1, i2, q0, q1, q2, t0, t1, t2):
        c = lax.axis_index("c")
        s = lax.axis_index("s")
        wid = c * SUB + s
        S = (s0, s1, s2)
        Dd = (d0, d1, d2)
        Gb = (g0, g1, g2)
        semI = (i0, i1, i2)
        semG = (q0, q1, q2)
        semS = (t0, t1, t2)

        pltpu.sync_copy(y_hbm.at[pl.ds(s * TSLAB, TSLAB), :], ytab.at[pl.ds(s * TSLAB, TSLAB)])
        pltpu.sync_copy(y_hbm.at[pl.ds(s * TSLAB, TSLAB), :], ztab.at[pl.ds(s * TSLAB, TSLAB)])
        plsc.subcore_barrier()

        def idx_load(j, k):
            jw = jnp.where(j < ECH, j, 0)
            pltpu.async_copy(src_hbm.at[m, wid, jw], S[k], semI[k])
            pltpu.async_copy(dst_hbm.at[m, wid, jw], Dd[k], semI[k])

        def wait_idx(k):
            pltpu.make_async_copy(src_hbm.at[m, wid, 0], S[k], semI[k]).wait()
            pltpu.make_async_copy(dst_hbm.at[m, wid, 0], Dd[k], semI[k]).wait()

        def step(j, jj, first):
            X, Y, Z = jj % 3, (jj + 1) % 3, (jj + 2) % 3
            pltpu.make_async_copy(ytab.at[S[X]], Gb[X], semG[X]).wait()
            pltpu.async_copy(Gb[X], ztab.at[Dd[X]], semS[X], add=True)
            if not first:
                pltpu.make_async_copy(Gb[Z], ztab.at[Dd[Z]], semS[Z]).wait()
            idx_load(j + 2, Z)
            wait_idx(Y)
            pltpu.async_copy(ytab.at[S[Y]], Gb[Y], semG[Y])

        pltpu.sync_copy(src_hbm.at[m, wid, 0], s0)
        pltpu.sync_copy(dst_hbm.at[m, wid, 0], d0)
        pltpu.async_copy(ytab.at[s0], g0, semG[0])
        idx_load(jnp.int32(1), 1)
        step(jnp.int32(0), 0, True)
        step(jnp.int32(1), 1, False)
        step(jnp.int32(2), 2, False)

        def body(t, carry):
            j = t * 3
            step(j, 0, False)
            step(j + 1, 1, False)
            step(j + 2, 2, False)
            return carry

        lax.fori_loop(1, ECH // 3, body, 0)
        pltpu.make_async_copy(Gb[2], ztab.at[Dd[2]], semS[2]).wait()
        pltpu.make_async_copy(ytab.at[S[0]], Gb[0], semG[0]).wait()
        wait_idx(1)
        plsc.subcore_barrier()
        pltpu.sync_copy(ztab.at[pl.ds(s * TSLAB, TSLAB)], z_out.at[c, pl.ds(s * TSLAB, TSLAB), :])
        plsc.subcore_barrier()

    return _edge4_kernel


_edge4_m = tuple(_make_edge4_kernel(m) for m in range(2))


@functools.partial(
    pl.kernel,
    out_type=jax.ShapeDtypeStruct((2, CORES, B_PAD, PW), _F32),
    scratch_types=[
        pltpu.VMEM_SHARED((B_PAD, PW), _F32),
        pltpu.VMEM((PPW, PW), _F32),
        pltpu.VMEM((PC,), jnp.int32),
    ],
    **_MESH,
)
def _pool_kernel(h0_hbm, h1_hbm, batch_hbm, zeros_hbm, pout, ptab, rowbuf, bidx):
    c = lax.axis_index("c")
    s = lax.axis_index("s")
    wid = c * SUB + s
    for m in range(2):
        h_hbm = (h0_hbm, h1_hbm)[m]

        @pl.when(s == 0)
        def _():
            pltpu.sync_copy(zeros_hbm, ptab)

        pltpu.sync_copy(h_hbm.at[pl.ds(wid * PPW, PPW), :], rowbuf)
        plsc.subcore_barrier()

        def body(j, carry):
            pltpu.sync_copy(batch_hbm.at[m, wid, j], bidx)
            pltpu.sync_copy(rowbuf.at[pl.ds(j * PC, PC), :], ptab.at[bidx], add=True)
            return carry

        lax.fori_loop(0, PCH, body, 0)
        plsc.subcore_barrier()

        @pl.when(s == 0)
        def _():
            pltpu.sync_copy(ptab, pout.at[m, c])

        plsc.subcore_barrier()


def _dinv_of(dblk):
    return lax.rsqrt(dblk[0, :, 0:1] + dblk[1, :, 0:1] - 1.0)


_BLK = 512


def _tc1_body(x_ref, deg_ref, w_ref, o_ref):
    dinv = _dinv_of(deg_ref[0])
    y = jnp.dot(x_ref[0], w_ref[...], preferred_element_type=_F32, precision=_PREC) * dinv
    o_ref[0] = y[:, :64]
    o_ref[1] = y[:, 64:]


def _tc1(x, degp, w, m):
    return pl.pallas_call(
        _tc1_body,
        grid=(N_PAD // _BLK,),
        in_specs=[
            pl.BlockSpec((1, _BLK, D), lambda i: (m, i, 0)),
            pl.BlockSpec((1, CORES, _BLK, 16), lambda i: (m, 0, i, 0)),
            pl.BlockSpec((D, H), lambda i: (0, 0)),
        ],
        out_specs=pl.BlockSpec((CORES, _BLK, 64), lambda i: (0, i, 0)),
        out_shape=jax.ShapeDtypeStruct((CORES, N_PAD, 64), _F32),
    )(x, degp, w)


def _make_tcmid(hin, hout):
    pin, pout = hin // 2, hout // 2

    def body(z_ref, deg_ref, b_ref, w_ref, o_ref):
        z = jnp.concatenate([z_ref[0], z_ref[1]], axis=1)
        dinv = _dinv_of(deg_ref[0])
        h = jnp.maximum(z * dinv + b_ref[...], 0.0)
        y = jnp.dot(h, w_ref[...], preferred_element_type=_F32, precision=_PREC) * dinv
        o_ref[0] = y[:, :pout]
        o_ref[1] = y[:, pout:]

    def call(z, degp, b, w, m):
        return pl.pallas_call(
            body,
            grid=(N_PAD // _BLK,),
            in_specs=[
                pl.BlockSpec((CORES, _BLK, pin), lambda i: (0, i, 0)),
                pl.BlockSpec((1, CORES, _BLK, 16), lambda i: (m, 0, i, 0)),
                pl.BlockSpec((1, hin), lambda i: (0, 0)),
                pl.BlockSpec((hin, hout), lambda i: (0, 0)),
            ],
            out_specs=pl.BlockSpec((CORES, _BLK, pout), lambda i: (0, i, 0)),
            out_shape=jax.ShapeDtypeStruct((CORES, N_PAD, pout), _F32),
        )(z, degp, b, w)

    return call


_tcmid_128 = _make_tcmid(H, H)


def _tcmid_l4(z, degp, b, w, m):
    def body(z_ref, deg_ref, b_ref, w_ref, o_ref):
        zc = jnp.concatenate([z_ref[0], z_ref[1]], axis=1)
        dinv = _dinv_of(deg_ref[0])
        h = jnp.maximum(zc * dinv + b_ref[...], 0.0)
        o_ref[...] = jnp.dot(h, w_ref[...], preferred_element_type=_F32, precision=_PREC) * dinv

    return pl.pallas_call(
        body,
        grid=(N_PAD // _BLK,),
        in_specs=[
            pl.BlockSpec((CORES, _BLK, H // 2), lambda i: (0, i, 0)),
            pl.BlockSpec((1, CORES, _BLK, 16), lambda i: (m, 0, i, 0)),
            pl.BlockSpec((1, H), lambda i: (0, 0)),
            pl.BlockSpec((H, H // 2), lambda i: (0, 0)),
        ],
        out_specs=pl.BlockSpec((_BLK, H // 2), lambda i: (i, 0)),
        out_shape=jax.ShapeDtypeStruct((N_PAD, H // 2), _F32),
    )(z, degp, b, w)


def _stage_body(z_ref, y_ref, deg_ref, b_ref, o_ref):
    z = z_ref[0] + z_ref[1] - y_ref[...]
    dinv = _dinv_of(deg_ref[0])
    h = jnp.maximum(z * dinv + b_ref[...], 0.0)
    onecol = (lax.broadcasted_iota(jnp.int32, (_BLK, 16), 1) == 0).astype(_F32)
    o_ref[...] = jnp.concatenate([h, onecol], axis=1)


def _tc_stage(z, y4, degp, b, m):
    return pl.pallas_call(
        _stage_body,
        grid=(N_PAD // _BLK,),
        in_specs=[
            pl.BlockSpec((CORES, _BLK, 64), lambda i: (0, i, 0)),
            pl.BlockSpec((_BLK, 64), lambda i: (i, 0)),
            pl.BlockSpec((1, CORES, _BLK, 16), lambda i: (m, 0, i, 0)),
            pl.BlockSpec((1, H // 2), lambda i: (0, 0)),
        ],
        out_specs=pl.BlockSpec((_BLK, PW), lambda i: (i, 0)),
        out_shape=jax.ShapeDtypeStruct((N_PAD, PW), _F32),
    )(z, y4, degp, b)


def _tail_body(p_ref, n1_ref, n2_ref, wn1, bn1, wn2, bn2, wc1, bc1, wc2, bc2, wc3, bc3, o_ref):
    p = p_ref[...]
    t1 = p[0, 0] + p[0, 1]
    t2 = p[1, 0] + p[1, 1]
    c1 = jnp.sum(t1[:B, 64:], axis=1, keepdims=True)
    c2 = jnp.sum(t2[:B, 64:], axis=1, keepdims=True)
    g1 = t1[:B, :64] / jnp.maximum(c1, 1.0)
    g2 = t2[:B, :64] / jnp.maximum(c2, 1.0)

    def mm(a, w, b):
        return jnp.dot(a, w[...], preferred_element_type=_F32, precision=_PREC) + b[...]

    m1 = jnp.maximum(mm(jnp.maximum(mm(n1_ref[...], wn1, bn1), 0.0), wn2, bn2), 0.0)
    m2 = jnp.maximum(mm(jnp.maximum(mm(n2_ref[...], wn1, bn1), 0.0), wn2, bn2), 0.0)
    cat = jnp.concatenate([g1, g2, m1, m2], axis=1)
    h = jnp.maximum(mm(cat, wc1, bc1), 0.0) * _INV_BN
    h = jnp.maximum(mm(h, wc2, bc2), 0.0) * _INV_BN
    o_ref[...] = mm(h, wc3, bc3)


def _tc_tail(pp, n1, n2, p):
    args = (pp, n1, n2,
            p["Wn1"], p["bn1"].reshape(1, -1), p["Wn2"], p["bn2"].reshape(1, -1),
            p["Wc1"], p["bc1"].reshape(1, -1), p["Wc2"], p["bc2"].reshape(1, -1),
            p["Wc3"], p["bc3"].reshape(1, -1))
    return pl.pallas_call(
        _tail_body,
        out_shape=jax.ShapeDtypeStruct((B, L), _F32),
    )(*args)


def kernel(mol1_x, mol1_edge_index, mol1_batch, mol2_x, mol2_edge_index, mol2_batch,
           mol1_notes, mol2_notes, params):
    def pad_rows(x):
        return jnp.concatenate([x, jnp.zeros((N_PAD - N, x.shape[1]), _F32)])

    def pad_idx(i, fill):
        return jnp.concatenate([i, jnp.full((E_PAD - E,), fill, jnp.int32)])

    x = jnp.stack([pad_rows(mol1_x), pad_rows(mol2_x)])
    src = jnp.stack([pad_idx(mol1_edge_index[0], N_TAB - 1),
                     pad_idx(mol2_edge_index[0], N_TAB - 1)])
    dst = jnp.stack([pad_idx(mol1_edge_index[1], N_TAB - 1),
                     pad_idx(mol2_edge_index[1], N_TAB - 1)])
    src16 = src.reshape(2, SUB, EC, CHUNK)
    dst16 = dst.reshape(2, SUB, EC, CHUNK)
    src32e = src.reshape(2, NW, EC // 2, CHUNK)
    dst32e = dst.reshape(2, NW, EC // 2, CHUNK)
    dst32 = dst.reshape(2, NW, DC, CHUNK)
    batch = jnp.stack([
        jnp.concatenate([mol1_batch, jnp.full((N_PAD - N,), B, jnp.int32)]),
        jnp.concatenate([mol2_batch, jnp.full((N_PAD - N,), B, jnp.int32)]),
    ]).reshape(2, NW, PCH, PC)
    ones16 = jnp.ones((N_PAD, 16), _F32)
    pzeros = jnp.zeros((B_PAD, PW), _F32)

    degp = _deg_kernel(dst32, ones16)
    h4s = []
    for m in range(2):
        y = _tc1(x, degp, params["W1"], m)
        z = _edge64_m[m](y, src16, dst16)
        y = _tcmid_128(z, degp, params["b1"].reshape(1, -1), params["W2"], m)
        z = _edge64_m[m](y, src16, dst16)
        y = _tcmid_128(z, degp, params["b2"].reshape(1, -1), params["W3"], m)
        z = _edge64_m[m](y, src16, dst16)
        y4 = _tcmid_l4(z, degp, params["b3"].reshape(1, -1), params["W4"], m)
        z4 = _edge4_m[m](y4, src32e, dst32e)
        h4s.append(_tc_stage(z4, y4, degp, params["b4"].reshape(1, -1), m))
    pp = _pool_kernel(h4s[0], h4s[1], batch, pzeros)
    return _tc_tail(pp, mol1_notes, mol2_notes, params)


# speedup vs baseline: 1.0034x; 1.0034x over previous
import functools
import math

import jax
import jax.numpy as jnp
from jax import lax
from jax.experimental import pallas as pl
from jax.experimental.pallas import tpu as pltpu
from jax.experimental.pallas import tpu_sc as plsc

N = 10000
E = 320000
D = 128
H = 128
L = 128
B = 512

CORES = 2
SUB = 16
NW = CORES * SUB

N_PAD = 10240
SLAB = N_PAD // SUB
N_TAB = 10112
TSLAB = N_TAB // SUB
CHUNK = 128
E_PAD = 331776
EC = E_PAD // SUB // CHUNK
DC = E_PAD // NW // CHUNK
B_PAD = 520
PW = 80
PC = 64
PPW = N_PAD // NW
PCH = PPW // PC

_MESH = dict(mesh=plsc.VectorSubcoreMesh(core_axis_name="c", subcore_axis_name="s"))
_F32 = jnp.float32
_INV_BN = 1.0 / math.sqrt(1.0 + 1e-5)
_PREC = jax.lax.Precision.HIGHEST


@functools.partial(
    pl.kernel,
    out_type=jax.ShapeDtypeStruct((2, CORES, N_PAD, 16), _F32),
    scratch_types=[
        pltpu.VMEM_SHARED((N_PAD, 16), _F32),
        pltpu.VMEM_SHARED((N_PAD, 16), _F32),
        pltpu.VMEM((CHUNK,), jnp.int32),
        pltpu.VMEM((CHUNK,), jnp.int32),
        pltpu.VMEM((CHUNK,), jnp.int32),
        pltpu.VMEM((CHUNK,), jnp.int32),
        pltpu.VMEM((CHUNK,), jnp.int32),
        pltpu.VMEM((CHUNK,), jnp.int32),
        pltpu.VMEM((CHUNK, 16), _F32),
        pltpu.SemaphoreType.DMA,
        pltpu.SemaphoreType.DMA,
        pltpu.SemaphoreType.DMA,
        pltpu.SemaphoreType.DMA,
        pltpu.SemaphoreType.DMA,
        pltpu.SemaphoreType.DMA,
    ],
    **_MESH,
)
def _deg_kernel(dst_hbm, ones_hbm, deg_out, degA, degB,
                a0, a1, a2, b0, b1, b2, onesbuf, i0, i1, i2, t0, t1, t2):
    c = lax.axis_index("c")
    s = lax.axis_index("s")
    wid = c * SUB + s
    Ea = (a0, a1, a2)
    Eb = (b0, b1, b2)
    semI = (i0, i1, i2)
    semS = (t0, t1, t2)
    pltpu.sync_copy(ones_hbm.at[pl.ds(0, CHUNK), :], onesbuf)
    pltpu.sync_copy(ones_hbm.at[pl.ds(s * SLAB, SLAB), :], degA.at[pl.ds(s * SLAB, SLAB)])
    pltpu.sync_copy(ones_hbm.at[pl.ds(s * SLAB, SLAB), :], degB.at[pl.ds(s * SLAB, SLAB)])
    plsc.subcore_barrier()

    def idx_load(j, k):
        jw = jnp.where(j < DC, j, 0)
        pltpu.async_copy(dst_hbm.at[0, wid, jw], Ea[k], semI[k])
        pltpu.async_copy(dst_hbm.at[1, wid, jw], Eb[k], semI[k])

    def wait_idx(k):
        pltpu.make_async_copy(dst_hbm.at[0, wid, 0], Ea[k], semI[k]).wait()
        pltpu.make_async_copy(dst_hbm.at[1, wid, 0], Eb[k], semI[k]).wait()

    def wait_scat(k):
        pltpu.make_async_copy(onesbuf, degA.at[Ea[k]], semS[k]).wait()
        pltpu.make_async_copy(onesbuf, degB.at[Eb[k]], semS[k]).wait()

    def step(j, jj, first):
        X, Z = jj % 3, (jj + 2) % 3
        if not first:
            wait_idx(X)
        pltpu.async_copy(onesbuf, degA.at[Ea[X]], semS[X], add=True)
        pltpu.async_copy(onesbuf, degB.at[Eb[X]], semS[X], add=True)
        if not first:
            wait_scat(Z)
        idx_load(j + 2, Z)

    pltpu.sync_copy(dst_hbm.at[0, wid, 0], a0)
    pltpu.sync_copy(dst_hbm.at[1, wid, 0], b0)
    idx_load(jnp.int32(1), 1)
    step(jnp.int32(0), 0, True)
    step(jnp.int32(1), 1, False)
    step(jnp.int32(2), 2, False)

    def body(t, carry):
        j = t * 3
        step(j, 0, False)
        step(j + 1, 1, False)
        step(j + 2, 2, False)
        return carry

    lax.fori_loop(1, DC // 3, body, 0)
    wait_scat((DC - 1) % 3)
    wait_idx(DC % 3)
    wait_idx((DC + 1) % 3)
    plsc.subcore_barrier()
    pltpu.sync_copy(degA.at[pl.ds(s * SLAB, SLAB)], deg_out.at[0, c, pl.ds(s * SLAB, SLAB), :])
    pltpu.sync_copy(degB.at[pl.ds(s * SLAB, SLAB)], deg_out.at[1, c, pl.ds(s * SLAB, SLAB), :])


def _make_edge_kernel(P, m):
    @functools.partial(
        pl.kernel,
        out_type=jax.ShapeDtypeStruct((CORES, N_PAD, P), _F32),
        scratch_types=[
            pltpu.VMEM_SHARED((N_TAB, P), _F32),
            pltpu.VMEM_SHARED((N_TAB, P), _F32),
            pltpu.VMEM((CHUNK,), jnp.int32),
            pltpu.VMEM((CHUNK,), jnp.int32),
            pltpu.VMEM((CHUNK,), jnp.int32),
            pltpu.VMEM((CHUNK,), jnp.int32),
            pltpu.VMEM((CHUNK,), jnp.int32),
            pltpu.VMEM((CHUNK,), jnp.int32),
            pltpu.VMEM((CHUNK, P), _F32),
            pltpu.VMEM((CHUNK, P), _F32),
            pltpu.VMEM((CHUNK, P), _F32),
            pltpu.SemaphoreType.DMA,
            pltpu.SemaphoreType.DMA,
            pltpu.SemaphoreType.DMA,
            pltpu.SemaphoreType.DMA,
            pltpu.SemaphoreType.DMA,
            pltpu.SemaphoreType.DMA,
            pltpu.SemaphoreType.DMA,
            pltpu.SemaphoreType.DMA,
            pltpu.SemaphoreType.DMA,
        ],
        **_MESH,
    )
    def _edge_kernel(y_hbm, src_hbm, dst_hbm, z_out, ytab, ztab,
                     s0, s1, s2, d0, d1, d2, g0, g1, g2,
                     i0, i1, i2, q0, q1, q2, t0, t1, t2):
        c = lax.axis_index("c")
        s = lax.axis_index("s")
        S = (s0, s1, s2)
        Dd = (d0, d1, d2)
        Gb = (g0, g1, g2)
        semI = (i0, i1, i2)
        semG = (q0, q1, q2)
        semS = (t0, t1, t2)

        if True:
            pltpu.sync_copy(y_hbm.at[c, pl.ds(s * TSLAB, TSLAB), :], ytab.at[pl.ds(s * TSLAB, TSLAB)])
            pltpu.sync_copy(y_hbm.at[c, pl.ds(s * TSLAB, TSLAB), :], ztab.at[pl.ds(s * TSLAB, TSLAB)])
            plsc.subcore_barrier()

            def idx_load(j, k):
                jw = jnp.where(j < EC, j, 0)
                pltpu.async_copy(src_hbm.at[m, s, jw], S[k], semI[k])
                pltpu.async_copy(dst_hbm.at[m, s, jw], Dd[k], semI[k])

            def wait_idx(k):
                pltpu.make_async_copy(src_hbm.at[m, s, 0], S[k], semI[k]).wait()
                pltpu.make_async_copy(dst_hbm.at[m, s, 0], Dd[k], semI[k]).wait()

            def step(j, jj, first):
                X, Y, Z = jj % 3, (jj + 1) % 3, (jj + 2) % 3
                pltpu.make_async_copy(ytab.at[S[X]], Gb[X], semG[X]).wait()
                pltpu.async_copy(Gb[X], ztab.at[Dd[X]], semS[X], add=True)
                if first:
                    pass
                else:
                    pltpu.make_async_copy(Gb[Z], ztab.at[Dd[Z]], semS[Z]).wait()
                idx_load(j + 2, Z)
                wait_idx(Y)
                pltpu.async_copy(ytab.at[S[Y]], Gb[Y], semG[Y])

            pltpu.sync_copy(src_hbm.at[m, s, 0], s0)
            pltpu.sync_copy(dst_hbm.at[m, s, 0], d0)
            pltpu.async_copy(ytab.at[s0], g0, semG[0])
            idx_load(jnp.int32(1), 1)

            step(jnp.int32(0), 0, True)
            step(jnp.int32(1), 1, False)
            step(jnp.int32(2), 2, False)

            def body(t, carry):
                j = t * 3
                step(j, 0, False)
                step(j + 1, 1, False)
                step(j + 2, 2, False)
                return carry

            lax.fori_loop(1, EC // 3, body, 0)

            pltpu.make_async_copy(Gb[2], ztab.at[Dd[2]], semS[2]).wait()
            pltpu.make_async_copy(ytab.at[S[0]], Gb[0], semG[0]).wait()
            wait_idx(1)
            plsc.subcore_barrier()
            pltpu.sync_copy(ztab.at[pl.ds(s * TSLAB, TSLAB)], z_out.at[c, pl.ds(s * TSLAB, TSLAB), :])
            plsc.subcore_barrier()

    return _edge_kernel


_edge64_m = tuple(_make_edge_kernel(64, m) for m in range(2))
_edge32_m = tuple(_make_edge_kernel(32, m) for m in range(2))


@functools.partial(
    pl.kernel,
    out_type=jax.ShapeDtypeStruct((2, CORES, B_PAD, PW), _F32),
    scratch_types=[
        pltpu.VMEM_SHARED((B_PAD, PW), _F32),
        pltpu.VMEM((PPW, PW), _F32),
        pltpu.VMEM((PC,), jnp.int32),
    ],
    **_MESH,
)
def _pool_kernel(h0_hbm, h1_hbm, batch_hbm, zeros_hbm, pout, ptab, rowbuf, bidx):
    c = lax.axis_index("c")
    s = lax.axis_index("s")
    wid = c * SUB + s
    for m in range(2):
        h_hbm = (h0_hbm, h1_hbm)[m]

        @pl.when(s == 0)
        def _():
            pltpu.sync_copy(zeros_hbm, ptab)

        pltpu.sync_copy(h_hbm.at[pl.ds(wid * PPW, PPW), :], rowbuf)
        plsc.subcore_barrier()

        def body(j, carry):
            pltpu.sync_copy(batch_hbm.at[m, wid, j], bidx)
            pltpu.sync_copy(rowbuf.at[pl.ds(j * PC, PC), :], ptab.at[bidx], add=True)
            return carry

        lax.fori_loop(0, PCH, body, 0)
        plsc.subcore_barrier()

        @pl.when(s == 0)
        def _():
            pltpu.sync_copy(ptab, pout.at[m, c])

        plsc.subcore_barrier()


def _dinv_of(dblk):
    return lax.rsqrt(dblk[0, :, 0:1] + dblk[1, :, 0:1] - 1.0)


_BLK = 512


def _tc1_body(x_ref, deg_ref, w_ref, o_ref):
    dinv = _dinv_of(deg_ref[0])
    y = jnp.dot(x_ref[0], w_ref[...], preferred_element_type=_F32, precision=_PREC) * dinv
    o_ref[0] = y[:, :64]
    o_ref[1] = y[:, 64:]


def _tc1(x, degp, w, m):
    return pl.pallas_call(
        _tc1_body,
        grid=(N_PAD // _BLK,),
        in_specs=[
            pl.BlockSpec((1, _BLK, D), lambda i: (m, i, 0)),
            pl.BlockSpec((1, CORES, _BLK, 16), lambda i: (m, 0, i, 0)),
            pl.BlockSpec((D, H), lambda i: (0, 0)),
        ],
        out_specs=pl.BlockSpec((CORES, _BLK, 64), lambda i: (0, i, 0)),
        out_shape=jax.ShapeDtypeStruct((CORES, N_PAD, 64), _F32),
    )(x, degp, w)


def _make_tcmid(hin, hout):
    pin, pout = hin // 2, hout // 2

    def body(z_ref, deg_ref, b_ref, w_ref, o_ref):
        z = jnp.concatenate([z_ref[0], z_ref[1]], axis=1)
        dinv = _dinv_of(deg_ref[0])
        h = jnp.maximum(z * dinv + b_ref[...], 0.0)
        y = jnp.dot(h, w_ref[...], preferred_element_type=_F32, precision=_PREC) * dinv
        o_ref[0] = y[:, :pout]
        o_ref[1] = y[:, pout:]

    def call(z, degp, b, w, m):
        return pl.pallas_call(
            body,
            grid=(N_PAD // _BLK,),
            in_specs=[
                pl.BlockSpec((CORES, _BLK, pin), lambda i: (0, i, 0)),
                pl.BlockSpec((1, CORES, _BLK, 16), lambda i: (m, 0, i, 0)),
                pl.BlockSpec((1, hin), lambda i: (0, 0)),
                pl.BlockSpec((hin, hout), lambda i: (0, 0)),
            ],
            out_specs=pl.BlockSpec((CORES, _BLK, pout), lambda i: (0, i, 0)),
            out_shape=jax.ShapeDtypeStruct((CORES, N_PAD, pout), _F32),
        )(z, degp, b, w)

    return call


_tcmid_128 = _make_tcmid(H, H)
_tcmid_l4 = _make_tcmid(H, H // 2)


def _stage_body(z_ref, deg_ref, b_ref, o_ref):
    z = jnp.concatenate([z_ref[0], z_ref[1]], axis=1)
    dinv = _dinv_of(deg_ref[0])
    h = jnp.maximum(z * dinv + b_ref[...], 0.0)
    onecol = (lax.broadcasted_iota(jnp.int32, (_BLK, 16), 1) == 0).astype(_F32)
    o_ref[...] = jnp.concatenate([h, onecol], axis=1)


def _tc_stage(z, degp, b, m):
    return pl.pallas_call(
        _stage_body,
        grid=(N_PAD // _BLK,),
        in_specs=[
            pl.BlockSpec((CORES, _BLK, 32), lambda i: (0, i, 0)),
            pl.BlockSpec((1, CORES, _BLK, 16), lambda i: (m, 0, i, 0)),
            pl.BlockSpec((1, H // 2), lambda i: (0, 0)),
        ],
        out_specs=pl.BlockSpec((_BLK, PW), lambda i: (i, 0)),
        out_shape=jax.ShapeDtypeStruct((N_PAD, PW), _F32),
    )(z, degp, b)


def _tail_body(p_ref, n1_ref, n2_ref, wn1, bn1, wn2, bn2, wc1, bc1, wc2, bc2, wc3, bc3, o_ref):
    p = p_ref[...]
    t1 = p[0, 0] + p[0, 1]
    t2 = p[1, 0] + p[1, 1]
    c1 = jnp.sum(t1[:B, 64:], axis=1, keepdims=True)
    c2 = jnp.sum(t2[:B, 64:], axis=1, keepdims=True)
    g1 = t1[:B, :64] / jnp.maximum(c1, 1.0)
    g2 = t2[:B, :64] / jnp.maximum(c2, 1.0)

    def mm(a, w, b):
        return jnp.dot(a, w[...], preferred_element_type=_F32, precision=_PREC) + b[...]

    m1 = jnp.maximum(mm(jnp.maximum(mm(n1_ref[...], wn1, bn1), 0.0), wn2, bn2), 0.0)
    m2 = jnp.maximum(mm(jnp.maximum(mm(n2_ref[...], wn1, bn1), 0.0), wn2, bn2), 0.0)
    cat = jnp.concatenate([g1, g2, m1, m2], axis=1)
    h = jnp.maximum(mm(cat, wc1, bc1), 0.0) * _INV_BN
    h = jnp.maximum(mm(h, wc2, bc2), 0.0) * _INV_BN
    o_ref[...] = mm(h, wc3, bc3)


def _tc_tail(pp, n1, n2, p):
    args = (pp, n1, n2,
            p["Wn1"], p["bn1"].reshape(1, -1), p["Wn2"], p["bn2"].reshape(1, -1),
            p["Wc1"], p["bc1"].reshape(1, -1), p["Wc2"], p["bc2"].reshape(1, -1),
            p["Wc3"], p["bc3"].reshape(1, -1))
    return pl.pallas_call(
        _tail_body,
        out_shape=jax.ShapeDtypeStruct((B, L), _F32),
    )(*args)


def kernel(mol1_x, mol1_edge_index, mol1_batch, mol2_x, mol2_edge_index, mol2_batch,
           mol1_notes, mol2_notes, params):
    def pad_rows(x):
        return jnp.concatenate([x, jnp.zeros((N_PAD - N, x.shape[1]), _F32)])

    def pad_idx(i, fill):
        return jnp.concatenate([i, jnp.full((E_PAD - E,), fill, jnp.int32)])

    x = jnp.stack([pad_rows(mol1_x), pad_rows(mol2_x)])
    src = jnp.stack([pad_idx(mol1_edge_index[0], N_TAB - 1),
                     pad_idx(mol2_edge_index[0], N_TAB - 1)])
    dst = jnp.stack([pad_idx(mol1_edge_index[1], N_TAB - 1),
                     pad_idx(mol2_edge_index[1], N_TAB - 1)])
    src16 = src.reshape(2, SUB, EC, CHUNK)
    dst16 = dst.reshape(2, SUB, EC, CHUNK)
    dst32 = dst.reshape(2, NW, DC, CHUNK)
    batch = jnp.stack([
        jnp.concatenate([mol1_batch, jnp.full((N_PAD - N,), B, jnp.int32)]),
        jnp.concatenate([mol2_batch, jnp.full((N_PAD - N,), B, jnp.int32)]),
    ]).reshape(2, NW, PCH, PC)
    ones16 = jnp.ones((N_PAD, 16), _F32)
    pzeros = jnp.zeros((B_PAD, PW), _F32)

    degp = _deg_kernel(dst32, ones16)
    h4s = []
    for m in range(2):
        y = _tc1(x, degp, params["W1"], m)
        z = _edge64_m[m](y, src16, dst16)
        y = _tcmid_128(z, degp, params["b1"].reshape(1, -1), params["W2"], m)
        z = _edge64_m[m](y, src16, dst16)
        y = _tcmid_128(z, degp, params["b2"].reshape(1, -1), params["W3"], m)
        z = _edge64_m[m](y, src16, dst16)
        y4 = _tcmid_l4(z, degp, params["b3"].reshape(1, -1), params["W4"], m)
        z4 = _edge32_m[m](y4, src16, dst16)
        h4s.append(_tc_stage(z4, degp, params["b4"].reshape(1, -1), m))
    pp = _pool_kernel(h4s[0], h4s[1], batch, pzeros)
    return _tc_tail(pp, mol1_notes, mol2_notes, params)
